# baseline (speedup 1.0000x reference)
import jax
import jax.numpy as jnp
from jax.experimental import pallas as pl
from jax.experimental.pallas import tpu as pltpu

_TOP_K = 5
_BLK = 256


def _make_kernel(C, HW, blk):
    def _kern(x_ref, o_ref, wt_ref):
        b = pl.program_id(0)
        d = pl.program_id(1)
        rows0 = b * blk

        @pl.when(d == 0)
        def _():
            jcol = jax.lax.broadcasted_iota(jnp.int32, (blk, HW), 1).astype(
                jnp.float32)
            ri = (rows0 + jax.lax.broadcasted_iota(jnp.int32, (blk, 1), 0)
                  ).astype(jnp.float32)
            wt_ref[...] = 1.0 / (jnp.abs(jcol - ri) + 1e-05)

        sf = x_ref[...]
        cols = x_ref[:, pl.ds(rows0, blk)]
        sim = jax.lax.dot_general(
            cols, sf, (((0,), (0,)), ((), ())),
            preferred_element_type=jnp.float32)

        t = jnp.max(sim, axis=1, keepdims=True)
        for _ in range(_TOP_K - 1):
            t = jnp.max(jnp.where(sim < t, sim, -jnp.inf), axis=1,
                        keepdims=True)

        wmat = jnp.where(sim >= t, wt_ref[...], 0.0)

        out = jax.lax.dot_general(
            sf, wmat, (((1,), (1,)), ((), ())),
            preferred_element_type=jnp.float32)
        wsum = jax.lax.dot_general(
            jnp.ones((1, HW), jnp.float32), wmat, (((1,), (1,)), ((), ())),
            preferred_element_type=jnp.float32)
        o_ref[0] = out * (1.0 / wsum)

    return _kern


def kernel(feature_map):
    N, C, D, H, W = feature_map.shape
    HW = H * W
    assert N == 1
    x = feature_map.reshape(C, D * HW)

    blk = _BLK
    nb = HW // blk
    out = pl.pallas_call(
        _make_kernel(C, HW, blk),
        grid=(nb, D),
        in_specs=[pl.BlockSpec((C, HW), lambda b, d: (0, d))],
        out_specs=pl.BlockSpec((1, C, blk), lambda b, d: (d, 0, b)),
        out_shape=jax.ShapeDtypeStruct((D, C, HW), jnp.float32),
        scratch_shapes=[pltpu.VMEM((blk, HW), jnp.float32)],
    )(x)
    return out.reshape(N, C, D, H, W)


# speedup vs baseline: 1.0419x; 1.0419x over previous
import jax
import jax.numpy as jnp
from jax.experimental import pallas as pl
from jax.experimental.pallas import tpu as pltpu

_TOP_K = 5
_BLK = 512


def _make_kernel(C, HW, blk):
    def _kern(x_ref, o_ref, wt_ref):
        b = pl.program_id(0)
        d = pl.program_id(1)
        rows0 = b * blk

        @pl.when(d == 0)
        def _():
            jcol = jax.lax.broadcasted_iota(jnp.int32, (blk, HW), 1).astype(
                jnp.float32)
            ri = (rows0 + jax.lax.broadcasted_iota(jnp.int32, (blk, 1), 0)
                  ).astype(jnp.float32)
            wt_ref[...] = 1.0 / (jnp.abs(jcol - ri) + 1e-05)

        sf = x_ref[...]
        cols = x_ref[:, pl.ds(rows0, blk)]
        sim = jax.lax.dot_general(
            cols, sf, (((0,), (0,)), ((), ())),
            preferred_element_type=jnp.float32)

        t = jnp.max(sim, axis=1, keepdims=True)
        for _ in range(_TOP_K - 1):
            t = jnp.max(jnp.where(sim < t, sim, -jnp.inf), axis=1,
                        keepdims=True)

        wmat = jnp.where(sim >= t, wt_ref[...], 0.0)

        out = jax.lax.dot_general(
            sf, wmat, (((1,), (1,)), ((), ())),
            preferred_element_type=jnp.float32)
        wsum = jax.lax.dot_general(
            jnp.ones((1, HW), jnp.float32), wmat, (((1,), (1,)), ((), ())),
            preferred_element_type=jnp.float32)
        o_ref[0] = out * (1.0 / wsum)

    return _kern


def kernel(feature_map):
    N, C, D, H, W = feature_map.shape
    HW = H * W
    assert N == 1
    x = feature_map.reshape(C, D * HW)

    blk = _BLK
    nb = HW // blk
    out = pl.pallas_call(
        _make_kernel(C, HW, blk),
        grid=(nb, D),
        in_specs=[pl.BlockSpec((C, HW), lambda b, d: (0, d))],
        out_specs=pl.BlockSpec((1, C, blk), lambda b, d: (d, 0, b)),
        out_shape=jax.ShapeDtypeStruct((D, C, HW), jnp.float32),
        scratch_shapes=[pltpu.VMEM((blk, HW), jnp.float32)],
    )(x)
    return out.reshape(N, C, D, H, W)


# speedup vs baseline: 1.0843x; 1.0407x over previous
import jax
import jax.numpy as jnp
from jax.experimental import pallas as pl
from jax.experimental.pallas import tpu as pltpu

_TOP_K = 5
_BLK = 1024


def _make_kernel(C, HW, blk):
    def _kern(x_ref, o_ref, wt_ref):
        b = pl.program_id(0)
        d = pl.program_id(1)
        rows0 = b * blk

        @pl.when(d == 0)
        def _():
            jcol = jax.lax.broadcasted_iota(jnp.int32, (blk, HW), 1).astype(
                jnp.float32)
            ri = (rows0 + jax.lax.broadcasted_iota(jnp.int32, (blk, 1), 0)
                  ).astype(jnp.float32)
            wt_ref[...] = 1.0 / (jnp.abs(jcol - ri) + 1e-05)

        sf = x_ref[...]
        cols = x_ref[:, pl.ds(rows0, blk)]
        sim = jax.lax.dot_general(
            cols, sf, (((0,), (0,)), ((), ())),
            preferred_element_type=jnp.float32)

        t = jnp.max(sim, axis=1, keepdims=True)
        for _ in range(_TOP_K - 1):
            t = jnp.max(jnp.where(sim < t, sim, -jnp.inf), axis=1,
                        keepdims=True)

        wmat = jnp.where(sim >= t, wt_ref[...], 0.0)

        out = jax.lax.dot_general(
            sf, wmat, (((1,), (1,)), ((), ())),
            preferred_element_type=jnp.float32)
        wsum = jax.lax.dot_general(
            jnp.ones((1, HW), jnp.float32), wmat, (((1,), (1,)), ((), ())),
            preferred_element_type=jnp.float32)
        o_ref[0] = out * (1.0 / wsum)

    return _kern


def kernel(feature_map):
    N, C, D, H, W = feature_map.shape
    HW = H * W
    assert N == 1
    x = feature_map.reshape(C, D * HW)

    blk = _BLK
    nb = HW // blk
    out = pl.pallas_call(
        _make_kernel(C, HW, blk),
        grid=(nb, D),
        in_specs=[pl.BlockSpec((C, HW), lambda b, d: (0, d))],
        out_specs=pl.BlockSpec((1, C, blk), lambda b, d: (d, 0, b)),
        out_shape=jax.ShapeDtypeStruct((D, C, HW), jnp.float32),
        scratch_shapes=[pltpu.VMEM((blk, HW), jnp.float32)],
    )(x)
    return out.reshape(N, C, D, H, W)


# speedup vs baseline: 1.0853x; 1.0009x over previous
import jax
import jax.numpy as jnp
from jax.experimental import pallas as pl
from jax.experimental.pallas import tpu as pltpu

_TOP_K = 5
_BLK = 1024


def _make_kernel(C, HW, blk):
    def _kern(x_ref, o_ref, wt_ref):
        b = pl.program_id(0)
        d = pl.program_id(1)
        rows0 = b * blk

        @pl.when(d == 0)
        def _():
            jcol = jax.lax.broadcasted_iota(jnp.int32, (blk, HW), 1).astype(
                jnp.float32)
            ri = (rows0 + jax.lax.broadcasted_iota(jnp.int32, (blk, 1), 0)
                  ).astype(jnp.float32)
            wt_ref[...] = 1.0 / (jnp.abs(jcol - ri) + 1e-05)

        sf = x_ref[...]
        cols = x_ref[:, pl.ds(rows0, blk)]
        sim = jax.lax.dot_general(
            cols, sf, (((0,), (0,)), ((), ())),
            preferred_element_type=jnp.float32)

        t = jnp.max(sim, axis=1, keepdims=True)
        for _ in range(_TOP_K - 1):
            t = jnp.max(jnp.where(sim < t, sim, -jnp.inf), axis=1,
                        keepdims=True)

        wmat = jnp.where(sim >= t, wt_ref[...], 0.0)

        out = jax.lax.dot_general(
            sf, wmat, (((1,), (1,)), ((), ())),
            preferred_element_type=jnp.float32)
        wsum = jax.lax.dot_general(
            jnp.ones((1, HW), jnp.float32), wmat, (((1,), (1,)), ((), ())),
            preferred_element_type=jnp.float32)
        o_ref[0] = out * (1.0 / wsum)

    return _kern


def kernel(feature_map):
    N, C, D, H, W = feature_map.shape
    HW = H * W
    assert N == 1
    x = feature_map.reshape(C, D * HW)

    blk = _BLK
    nb = HW // blk
    out = pl.pallas_call(
        _make_kernel(C, HW, blk),
        grid=(nb, D),
        in_specs=[pl.BlockSpec((C, HW), lambda b, d: (0, d))],
        out_specs=pl.BlockSpec((1, C, blk), lambda b, d: (d, 0, b)),
        out_shape=jax.ShapeDtypeStruct((D, C, HW), jnp.float32),
        scratch_shapes=[pltpu.VMEM((blk, HW), jnp.float32)],
        compiler_params=pltpu.CompilerParams(
            dimension_semantics=("parallel", "arbitrary")),
    )(x)
    return out.reshape(N, C, D, H, W)
